# baseline (speedup 1.0000x reference)
import functools

import jax
import jax.numpy as jnp
from jax import lax
from jax.experimental import pallas as pl
from jax.experimental.pallas import tpu as pltpu
from jax.experimental.pallas import tpu_sc as plsc

_E = 8
_D = 1024
_F = 512
_T = 2048
_G = 256
_NT = _T // _G
_S = _NT + _E - 1
_G_SHIFT = _G.bit_length() - 1
_FH = _F // 2


def _router_body(x_ref, rw_ref, pos_ref, meta_ref, off_ref, logit_ref):
    i = pl.program_id(0)
    logit_ref[pl.ds(i * _G, _G), :] = jnp.dot(
        x_ref[...], rw_ref[...], preferred_element_type=jnp.float32)

    @pl.when(i == _NT - 1)
    def _():
        logits = logit_ref[...]
        m = jnp.max(logits, axis=1, keepdims=True)
        cols = lax.broadcasted_iota(jnp.int32, (_T, _E), 1)
        amax = jnp.min(jnp.where(logits == m, cols, _E), axis=1,
                       keepdims=True)
        oh = (cols == amax).astype(jnp.float32)
        incl = oh
        k = 1
        while k < _T:
            shifted = jnp.concatenate(
                [jnp.zeros((k, _E), jnp.float32), incl[:_T - k]], axis=0)
            incl = incl + shifted
            k *= 2
        counts = incl[_T - 1:_T, :]
        tri = (lax.broadcasted_iota(jnp.int32, (_E, _E), 0) <
               lax.broadcasted_iota(jnp.int32, (_E, _E), 1)
               ).astype(jnp.float32)
        offv = jnp.dot(counts, tri, preferred_element_type=jnp.float32,
                       precision=lax.Precision.HIGHEST)
        rank = jnp.sum(oh * (incl - 1.0), axis=1, keepdims=True)
        base = jnp.sum(oh * offv, axis=1, keepdims=True)
        pos_ref[...] = (rank + base).astype(jnp.int32)

        counts_i = counts.astype(jnp.int32)
        off_i = offv.astype(jnp.int32)
        nonempty = counts_i > 0
        first_t = jnp.where(nonempty, off_i >> _G_SHIFT, 0)
        last_t = jnp.where(nonempty,
                           (off_i + counts_i - 1) >> _G_SHIFT, -1)
        nsteps = jnp.where(nonempty, last_t - first_t + 1, 0)
        tri_i = (lax.broadcasted_iota(jnp.int32, (_E, _E), 0) <=
                 lax.broadcasted_iota(jnp.int32, (_E, _E), 1)
                 ).astype(jnp.float32)
        cum = jnp.dot(nsteps.astype(jnp.float32), tri_i,
                      preferred_element_type=jnp.float32,
                      precision=lax.Precision.HIGHEST).astype(jnp.int32)
        prev = cum - nsteps
        adj = first_t - prev
        sidx16 = lax.broadcasted_iota(jnp.int32, (16, _E), 0)
        lane8 = lax.broadcasted_iota(jnp.int32, (16, _E), 1)
        se = jnp.sum(
            (sidx16 >= jnp.broadcast_to(cum, (16, _E))).astype(jnp.int32),
            axis=1, keepdims=True)
        oh_e = (lane8 == jnp.minimum(se, _E - 1)).astype(jnp.int32)
        a = jnp.sum(oh_e * jnp.broadcast_to(adj, (16, _E)), axis=1,
                    keepdims=True)
        scol = lax.broadcasted_iota(jnp.int32, (16, 1), 0)
        tile = jnp.where(se < _E, scol + a, _NT - 1)
        meta_ref[...] = jnp.concatenate([tile, se], axis=1)
        off_ref[...] = jnp.concatenate(
            [off_i, jnp.full((1, 4), _T, jnp.int32)], axis=1)


def _router_tc(xf, router_w):
    return pl.pallas_call(
        _router_body,
        grid=(_NT,),
        in_specs=[
            pl.BlockSpec((_G, _D), lambda i: (i, 0)),
            pl.BlockSpec((_D, _E), lambda i: (0, 0)),
        ],
        out_specs=(
            pl.BlockSpec((_T, 1), lambda i: (0, 0)),
            pl.BlockSpec((16, 2), lambda i: (0, 0)),
            pl.BlockSpec((1, _E + 4), lambda i: (0, 0)),
        ),
        out_shape=(
            jax.ShapeDtypeStruct((_T, 1), jnp.int32),
            jax.ShapeDtypeStruct((16, 2), jnp.int32),
            jax.ShapeDtypeStruct((1, _E + 4), jnp.int32),
        ),
        scratch_shapes=[pltpu.VMEM((_T, _E), jnp.float32)],
        compiler_params=pltpu.CompilerParams(
            dimension_semantics=("arbitrary",)),
    )(xf, router_w)


def _sc_permute(xf, pos):
    info = plsc.get_sparse_core_info()
    nw = info.num_cores * info.num_subcores
    ch = _T // nw
    mesh = plsc.VectorSubcoreMesh(core_axis_name="c", subcore_axis_name="s")

    @functools.partial(
        pl.kernel,
        mesh=mesh,
        out_type=jax.ShapeDtypeStruct((_T, _D), jnp.float32),
        scratch_types=[
            pltpu.VMEM((ch,), jnp.int32),
            pltpu.VMEM((ch, _D), jnp.float32),
            pltpu.SemaphoreType.DMA,
        ],
    )
    def k(x_hbm, pos_hbm, out_hbm, idx_v, rows_v, sem):
        wid = lax.axis_index("s") * info.num_cores + lax.axis_index("c")
        b = wid * ch
        pltpu.sync_copy(pos_hbm.at[pl.ds(b, ch)], idx_v)
        pltpu.sync_copy(x_hbm.at[pl.ds(b, ch)], rows_v)
        pltpu.async_copy(rows_v, out_hbm.at[idx_v], sem).wait()

    return k(xf, pos)


_D2 = _D // 2


def _gmm_body(meta_ref, off_ref, x_ref, w1_ref, w3_ref, w2a_ref, w2b_ref,
              out_ref, g_acc, u_acc):
    s = pl.program_id(0)
    k = pl.program_id(1)
    t = meta_ref[s, 0]
    e = meta_ref[s, 1]
    lo = off_ref[0, e]
    hi = off_ref[0, e + 1]
    gid = t * _G + lax.broadcasted_iota(jnp.int32, (_G, 1), 0)
    mask = jnp.logical_and(gid >= lo, gid < hi)

    @pl.when(k == 0)
    def _():
        xk = jnp.where(mask, x_ref[...], 0.0).astype(jnp.bfloat16)
        g_acc[...] = jnp.dot(xk, w1_ref[0].astype(jnp.bfloat16),
                             preferred_element_type=jnp.float32)
        u_acc[...] = jnp.dot(xk, w3_ref[0].astype(jnp.bfloat16),
                             preferred_element_type=jnp.float32)

    @pl.when(k == 1)
    def _():
        xk = jnp.where(mask, x_ref[...], 0.0).astype(jnp.bfloat16)
        g_acc[...] += jnp.dot(xk, w1_ref[0].astype(jnp.bfloat16),
                              preferred_element_type=jnp.float32)
        u_acc[...] += jnp.dot(xk, w3_ref[0].astype(jnp.bfloat16),
                              preferred_element_type=jnp.float32)

    @pl.when(k == 2)
    def _():
        g = g_acc[...]
        u = u_acc[...]
        h = (g * jax.nn.sigmoid(g) * u).astype(jnp.bfloat16)
        y = (jnp.dot(h[:, :_FH], w2a_ref[0].astype(jnp.bfloat16),
                     preferred_element_type=jnp.float32) +
             jnp.dot(h[:, _FH:], w2b_ref[0].astype(jnp.bfloat16),
                     preferred_element_type=jnp.float32))
        first = jnp.logical_or(
            s == 0, meta_ref[jnp.maximum(s - 1, 0), 0] != t)

        @pl.when(first)
        def _():
            out_ref[...] = y

        @pl.when(jnp.logical_not(first))
        def _():
            out_ref[...] += y


def _gmm_tc(routed, w1, w3, w2, meta, off_row):
    def _e(s, m):
        return jnp.minimum(m[s, 1], _E - 1)

    def _eprev(s, m):
        return jnp.minimum(m[jnp.maximum(s - 1, 0), 1], _E - 1)

    def _w13map(s, k, m, o):
        return (_e(s, m), jnp.minimum(k, 1), 0)

    def _w2amap(s, k, m, o):
        return (jnp.where(k >= 1, _e(s, m), _eprev(s, m)), 0, 0)

    def _w2bmap(s, k, m, o):
        return (jnp.where(k >= 2, _e(s, m), _eprev(s, m)), 1, 0)

    grid_spec = pltpu.PrefetchScalarGridSpec(
        num_scalar_prefetch=2,
        grid=(_S, 3),
        in_specs=[
            pl.BlockSpec((_G, _D2),
                         lambda s, k, m, o: (m[s, 0], jnp.minimum(k, 1))),
            pl.BlockSpec((1, _D2, _F), _w13map),
            pl.BlockSpec((1, _D2, _F), _w13map),
            pl.BlockSpec((1, _FH, _D), _w2amap),
            pl.BlockSpec((1, _FH, _D), _w2bmap),
        ],
        out_specs=pl.BlockSpec((_G, _D), lambda s, k, m, o: (m[s, 0], 0)),
        scratch_shapes=[
            pltpu.VMEM((_G, _F), jnp.float32),
            pltpu.VMEM((_G, _F), jnp.float32),
        ],
    )
    return pl.pallas_call(
        _gmm_body,
        grid_spec=grid_spec,
        out_shape=jax.ShapeDtypeStruct((_T, _D), jnp.float32),
        compiler_params=pltpu.CompilerParams(
            dimension_semantics=("arbitrary", "arbitrary")),
    )(meta, off_row, routed, w1, w3, w2, w2)


def kernel(x, router_w, w1, w2, w3):
    b, s, d = x.shape
    xf = x.reshape(b * s, d)

    pos2d, meta, off_row = _router_tc(xf, router_w)
    routed = _sc_permute(xf, pos2d.reshape(_T))
    out = _gmm_tc(routed, w1, w3, w2, meta, off_row)
    return out.reshape(b, s, d)


# speedup vs baseline: 1.3353x; 1.3353x over previous
import functools

import jax
import jax.numpy as jnp
from jax import lax
from jax.experimental import pallas as pl
from jax.experimental.pallas import tpu as pltpu
from jax.experimental.pallas import tpu_sc as plsc

_E = 8
_D = 1024
_F = 512
_T = 2048
_G = 512
_NT = _T // _G
_S = _NT + _E - 1
_G_SHIFT = _G.bit_length() - 1
_FH = _F // 2


def _router_body(x_ref, rw_ref, pos_ref, meta_ref, off_ref):
    if True:
        logits = jnp.dot(x_ref[...], rw_ref[...],
                         preferred_element_type=jnp.float32)
        m = jnp.max(logits, axis=1, keepdims=True)
        cols = lax.broadcasted_iota(jnp.int32, (_T, _E), 1)
        amax = jnp.min(jnp.where(logits == m, cols, _E), axis=1,
                       keepdims=True)
        oh = (cols == amax).astype(jnp.float32)
        incl = oh
        k = 1
        while k < _T:
            shifted = jnp.concatenate(
                [jnp.zeros((k, _E), jnp.float32), incl[:_T - k]], axis=0)
            incl = incl + shifted
            k *= 2
        counts = incl[_T - 1:_T, :]
        tri = (lax.broadcasted_iota(jnp.int32, (_E, _E), 0) <
               lax.broadcasted_iota(jnp.int32, (_E, _E), 1)
               ).astype(jnp.float32)
        offv = jnp.dot(counts, tri, preferred_element_type=jnp.float32,
                       precision=lax.Precision.HIGHEST)
        rank = jnp.sum(oh * (incl - 1.0), axis=1, keepdims=True)
        base = jnp.sum(oh * offv, axis=1, keepdims=True)
        pos_ref[...] = (rank + base).astype(jnp.int32)

        counts_i = counts.astype(jnp.int32)
        off_i = offv.astype(jnp.int32)
        nonempty = counts_i > 0
        first_t = jnp.where(nonempty, off_i >> _G_SHIFT, 0)
        last_t = jnp.where(nonempty,
                           (off_i + counts_i - 1) >> _G_SHIFT, -1)
        nsteps = jnp.where(nonempty, last_t - first_t + 1, 0)
        tri_i = (lax.broadcasted_iota(jnp.int32, (_E, _E), 0) <=
                 lax.broadcasted_iota(jnp.int32, (_E, _E), 1)
                 ).astype(jnp.float32)
        cum = jnp.dot(nsteps.astype(jnp.float32), tri_i,
                      preferred_element_type=jnp.float32,
                      precision=lax.Precision.HIGHEST).astype(jnp.int32)
        prev = cum - nsteps
        adj = first_t - prev
        sidx16 = lax.broadcasted_iota(jnp.int32, (16, _E), 0)
        lane8 = lax.broadcasted_iota(jnp.int32, (16, _E), 1)
        se = jnp.sum(
            (sidx16 >= jnp.broadcast_to(cum, (16, _E))).astype(jnp.int32),
            axis=1, keepdims=True)
        oh_e = (lane8 == jnp.minimum(se, _E - 1)).astype(jnp.int32)
        a = jnp.sum(oh_e * jnp.broadcast_to(adj, (16, _E)), axis=1,
                    keepdims=True)
        scol = lax.broadcasted_iota(jnp.int32, (16, 1), 0)
        tile = jnp.where(se < _E, scol + a, _NT - 1)
        meta_ref[...] = jnp.concatenate([tile, se], axis=1)
        off_ref[...] = jnp.concatenate(
            [off_i, jnp.full((1, 4), _T, jnp.int32)], axis=1)


def _router_tc(xf, router_w):
    return pl.pallas_call(
        _router_body,
        out_shape=(
            jax.ShapeDtypeStruct((_T, 1), jnp.int32),
            jax.ShapeDtypeStruct((16, 2), jnp.int32),
            jax.ShapeDtypeStruct((1, _E + 4), jnp.int32),
        ),
    )(xf, router_w)


def _sc_permute(xf, pos):
    info = plsc.get_sparse_core_info()
    nw = info.num_cores * info.num_subcores
    ch = _T // nw
    mesh = plsc.VectorSubcoreMesh(core_axis_name="c", subcore_axis_name="s")

    @functools.partial(
        pl.kernel,
        mesh=mesh,
        out_type=jax.ShapeDtypeStruct((_T, _D), jnp.float32),
        scratch_types=[
            pltpu.VMEM((ch,), jnp.int32),
            pltpu.VMEM((ch, _D), jnp.float32),
            pltpu.SemaphoreType.DMA,
        ],
    )
    def k(x_hbm, pos_hbm, out_hbm, idx_v, rows_v, sem):
        wid = lax.axis_index("s") * info.num_cores + lax.axis_index("c")
        b = wid * ch
        pltpu.sync_copy(pos_hbm.at[pl.ds(b, ch)], idx_v)
        pltpu.sync_copy(x_hbm.at[pl.ds(b, ch)], rows_v)
        pltpu.async_copy(rows_v, out_hbm.at[idx_v], sem).wait()

    return k(xf, pos)


def _gmm_body(meta_ref, off_ref, x_ref, w1_ref, w3_ref, w2_ref, out_ref):
    s = pl.program_id(0)
    t = meta_ref[s, 0]
    e = meta_ref[s, 1]
    lo = off_ref[0, e]
    hi = off_ref[0, e + 1]
    gid = t * _G + lax.broadcasted_iota(jnp.int32, (_G, 1), 0)
    mask = jnp.logical_and(gid >= lo, gid < hi)
    xm = jnp.where(mask, x_ref[...], 0.0).astype(jnp.bfloat16)
    g = jnp.dot(xm, w1_ref[0].astype(jnp.bfloat16),
                preferred_element_type=jnp.float32)
    u = jnp.dot(xm, w3_ref[0].astype(jnp.bfloat16),
                preferred_element_type=jnp.float32)
    h = (g * jax.nn.sigmoid(g) * u).astype(jnp.bfloat16)
    y = jnp.dot(h, w2_ref[0].astype(jnp.bfloat16),
                preferred_element_type=jnp.float32)
    first = jnp.logical_or(s == 0, meta_ref[jnp.maximum(s - 1, 0), 0] != t)

    @pl.when(first)
    def _():
        out_ref[...] = y

    @pl.when(jnp.logical_not(first))
    def _():
        out_ref[...] += y


def _gmm_tc(routed, w1, w3, w2, meta, off_row):
    def _wmap(s, m, o):
        return (jnp.minimum(m[s, 1], _E - 1), 0, 0)

    grid_spec = pltpu.PrefetchScalarGridSpec(
        num_scalar_prefetch=2,
        grid=(_S,),
        in_specs=[
            pl.BlockSpec((_G, _D), lambda s, m, o: (m[s, 0], 0)),
            pl.BlockSpec((1, _D, _F), _wmap),
            pl.BlockSpec((1, _D, _F), _wmap),
            pl.BlockSpec((1, _F, _D), _wmap),
        ],
        out_specs=pl.BlockSpec((_G, _D), lambda s, m, o: (m[s, 0], 0)),
    )
    return pl.pallas_call(
        _gmm_body,
        grid_spec=grid_spec,
        out_shape=jax.ShapeDtypeStruct((_T, _D), jnp.float32),
        compiler_params=pltpu.CompilerParams(
            dimension_semantics=("arbitrary",)),
    )(meta, off_row, routed, w1, w3, w2)


def kernel(x, router_w, w1, w2, w3):
    b, s, d = x.shape
    xf = x.reshape(b * s, d)

    pos2d, meta, off_row = _router_tc(xf, router_w)
    routed = _sc_permute(xf, pos2d.reshape(_T))
    out = _gmm_tc(routed, w1, w3, w2, meta, off_row)
    return out.reshape(b, s, d)
